# Initial kernel scaffold; baseline (speedup 1.0000x reference)
import jax
import jax.numpy as jnp
from jax.experimental import pallas as pl


def kernel(x, edge_index, x_batch, W0, b0, W1, b1):
    raise NotImplementedError("write your pallas kernel here")


# speedup vs baseline: 10.4048x; 10.4048x over previous
import functools

import jax
import jax.numpy as jnp
from jax import lax
from jax.experimental import pallas as pl
from jax.experimental.pallas import tpu as pltpu
from jax.experimental.pallas import tpu_sc as plsc

N = 10000
E = 160000
D = 256
H = 128
N_SUB = 16
ROW_CHUNK = 640
LAST_CHUNK = N - 15 * ROW_CHUNK
ZROWS = 80

DEG_E_PER_TILE = E // (2 * N_SUB)
DEG_BLK = 40
DEG_NBLK = DEG_E_PER_TILE // DEG_BLK

SC_E_PER_TILE = E // N_SUB
SC_BLK = 80
SC_NBLK = SC_E_PER_TILE // SC_BLK

_MESH = plsc.VectorSubcoreMesh(core_axis_name="c", subcore_axis_name="s")


def _zero_vmem(buf):
    rows, cols = buf.shape

    @pl.loop(0, rows)
    def _(i):
        @pl.loop(0, cols, step=16)
        def _(j):
            buf.at[pl.ds(i, 1), pl.ds(j, 16)][...] = jnp.zeros((1, 16), jnp.float32)


def _tile_rows(tile):
    return pl.multiple_of(tile * ROW_CHUNK, 8)


def _zero_acc_slice(acc, zbuf, tile):
    row0 = _tile_rows(tile)

    @pl.when(tile < 15)
    def _():
        @pl.loop(0, ROW_CHUNK, step=ZROWS)
        def _(m):
            pltpu.sync_copy(zbuf, acc.at[pl.ds(pl.multiple_of(row0 + m, 8), ZROWS)])

    @pl.when(tile == 15)
    def _():
        @pl.loop(0, LAST_CHUNK, step=ZROWS)
        def _(m):
            pltpu.sync_copy(zbuf, acc.at[pl.ds(pl.multiple_of(row0 + m, 8), ZROWS)])


def _copy_out_slice(acc, out_hbm, tile):
    row0 = _tile_rows(tile)

    @pl.when(tile < 15)
    def _():
        pltpu.sync_copy(acc.at[pl.ds(row0, ROW_CHUNK)],
                        out_hbm.at[pl.ds(row0, ROW_CHUNK)])

    @pl.when(tile == 15)
    def _():
        pltpu.sync_copy(acc.at[pl.ds(row0, LAST_CHUNK)],
                        out_hbm.at[pl.ds(row0, LAST_CHUNK)])


def _sc_degree(dst):
    @functools.partial(
        pl.kernel,
        out_type=(
            jax.ShapeDtypeStruct((N, H), jnp.float32),
            jax.ShapeDtypeStruct((N, H), jnp.float32),
        ),
        mesh=_MESH,
        scratch_types=[
            pltpu.VMEM_SHARED((N, H), jnp.float32),
            pltpu.VMEM((DEG_BLK,), jnp.int32),
            pltpu.VMEM((DEG_BLK, H), jnp.float32),
            pltpu.VMEM((ZROWS, H), jnp.float32),
        ],
    )
    def deg_kernel(dst_hbm, dega_hbm, degb_hbm, acc, idx, ones_buf, zbuf):
        c = lax.axis_index("c")
        t = lax.axis_index("s")

        _zero_vmem(zbuf)
        _zero_acc_slice(acc, zbuf, t)

        @pl.loop(0, DEG_BLK)
        def _(i):
            @pl.loop(0, H, step=16)
            def _(j):
                ones_buf.at[pl.ds(i, 1), pl.ds(j, 16)][...] = jnp.ones(
                    (1, 16), jnp.float32
                )

        plsc.subcore_barrier()

        base0 = c * (E // 2) + t * DEG_E_PER_TILE

        @pl.loop(0, DEG_NBLK)
        def _(k):
            pltpu.sync_copy(dst_hbm.at[pl.ds(base0 + k * DEG_BLK, DEG_BLK)], idx)
            pltpu.sync_copy(ones_buf, acc.at[idx], add=True)

        plsc.subcore_barrier()

        @pl.when(c == 0)
        def _():
            _copy_out_slice(acc, dega_hbm, t)

        @pl.when(c == 1)
        def _():
            _copy_out_slice(acc, degb_hbm, t)

    return deg_kernel(dst)


def _sc_scatter(ga, gb, src, dst):
    @functools.partial(
        pl.kernel,
        out_type=(
            jax.ShapeDtypeStruct((N, H), jnp.float32),
            jax.ShapeDtypeStruct((N, H), jnp.float32),
        ),
        mesh=_MESH,
        scratch_types=[
            pltpu.VMEM_SHARED((N, H), jnp.float32),
            pltpu.VMEM((SC_E_PER_TILE,), jnp.int32),
            pltpu.VMEM((SC_E_PER_TILE,), jnp.int32),
            pltpu.VMEM((SC_BLK,), jnp.int32),
            pltpu.VMEM((SC_BLK, H), jnp.float32),
            pltpu.VMEM((ZROWS, H), jnp.float32),
        ],
    )
    def scat_kernel(ga_hbm, gb_hbm, src_hbm, dst_hbm, sa_hbm, sb_hbm,
                    acc, src_all, dst_all, dst_idx, rows_buf, zbuf):
        c = lax.axis_index("c")
        t = lax.axis_index("s")

        _zero_vmem(zbuf)
        _zero_acc_slice(acc, zbuf, t)

        ebase = t * SC_E_PER_TILE
        pltpu.sync_copy(src_hbm.at[pl.ds(ebase, SC_E_PER_TILE)], src_all)
        pltpu.sync_copy(dst_hbm.at[pl.ds(ebase, SC_E_PER_TILE)], dst_all)

        plsc.subcore_barrier()

        def run(g_hbm, out_hbm):
            @pl.loop(0, SC_NBLK)
            def _(k):
                eoff = k * SC_BLK
                pltpu.sync_copy(g_hbm.at[src_all.at[pl.ds(eoff, SC_BLK)]], rows_buf)
                @pl.loop(0, SC_BLK, step=16)
                def _(j):
                    dst_idx.at[pl.ds(j, 16)][...] = dst_all.at[
                        pl.ds(eoff + j, 16)
                    ][...]
                pltpu.sync_copy(rows_buf, acc.at[dst_idx], add=True)

            plsc.subcore_barrier()
            _copy_out_slice(acc, out_hbm, t)

        @pl.when(c == 0)
        def _():
            run(ga_hbm, sa_hbm)

        @pl.when(c == 1)
        def _():
            run(gb_hbm, sb_hbm)

    return scat_kernel(ga, gb, src, dst)


def _tc_matmul(x, w):
    def body(x_ref, w_ref, h_ref):
        h_ref[...] = jnp.dot(x_ref[...], w_ref[...],
                             preferred_element_type=jnp.float32)

    return pl.pallas_call(
        body,
        grid=(10,),
        in_specs=[
            pl.BlockSpec((1000, D), lambda i: (i, 0)),
            pl.BlockSpec((D, D), lambda i: (0, 0)),
        ],
        out_specs=pl.BlockSpec((1000, D), lambda i: (i, 0)),
        out_shape=jax.ShapeDtypeStruct((N, D), jnp.float32),
    )(x, w)


def _dinv_block(da_ref, db_ref):
    deg = 1.0 + da_ref[:, 0:1] + db_ref[:, 0:1]
    return lax.rsqrt(deg)


def _tc_scale(h, dega, degb):
    def body(h_ref, da_ref, db_ref, ga_ref, gb_ref):
        g = h_ref[...] * _dinv_block(da_ref, db_ref)
        ga_ref[...] = g[:, :H]
        gb_ref[...] = g[:, H:]

    return pl.pallas_call(
        body,
        grid=(10,),
        in_specs=[
            pl.BlockSpec((1000, D), lambda i: (i, 0)),
            pl.BlockSpec((1000, H), lambda i: (i, 0)),
            pl.BlockSpec((1000, H), lambda i: (i, 0)),
        ],
        out_specs=[
            pl.BlockSpec((1000, H), lambda i: (i, 0)),
            pl.BlockSpec((1000, H), lambda i: (i, 0)),
        ],
        out_shape=[
            jax.ShapeDtypeStruct((N, H), jnp.float32),
            jax.ShapeDtypeStruct((N, H), jnp.float32),
        ],
    )(h, dega, degb)


def _tc_mid(sa, sb, ga, gb, dega, degb, b0, w1):
    b0r = b0.reshape(1, D)

    def body(sa_ref, sb_ref, ga_ref, gb_ref, da_ref, db_ref, b0_ref, w1_ref,
             g1a_ref, g1b_ref):
        dinv = _dinv_block(da_ref, db_ref)
        pre_a = (sa_ref[...] + ga_ref[...]) * dinv
        pre_b = (sb_ref[...] + gb_ref[...]) * dinv
        pre = jnp.concatenate([pre_a, pre_b], axis=1) + b0_ref[...]
        a = jnp.tanh(pre)
        h1 = jnp.dot(a, w1_ref[...], preferred_element_type=jnp.float32)
        g1 = h1 * dinv
        g1a_ref[...] = g1[:, :H]
        g1b_ref[...] = g1[:, H:]

    return pl.pallas_call(
        body,
        grid=(10,),
        in_specs=[
            pl.BlockSpec((1000, H), lambda i: (i, 0)),
            pl.BlockSpec((1000, H), lambda i: (i, 0)),
            pl.BlockSpec((1000, H), lambda i: (i, 0)),
            pl.BlockSpec((1000, H), lambda i: (i, 0)),
            pl.BlockSpec((1000, H), lambda i: (i, 0)),
            pl.BlockSpec((1000, H), lambda i: (i, 0)),
            pl.BlockSpec((1, D), lambda i: (0, 0)),
            pl.BlockSpec((D, D), lambda i: (0, 0)),
        ],
        out_specs=[
            pl.BlockSpec((1000, H), lambda i: (i, 0)),
            pl.BlockSpec((1000, H), lambda i: (i, 0)),
        ],
        out_shape=[
            jax.ShapeDtypeStruct((N, H), jnp.float32),
            jax.ShapeDtypeStruct((N, H), jnp.float32),
        ],
    )(sa, sb, ga, gb, dega, degb, b0r, w1)


def _tc_out(s1a, s1b, g1a, g1b, dega, degb, b1):
    b1r = b1.reshape(1, D)

    def body(sa_ref, sb_ref, ga_ref, gb_ref, da_ref, db_ref, b1_ref, o_ref):
        dinv = _dinv_block(da_ref, db_ref)
        oa = (sa_ref[...] + ga_ref[...]) * dinv
        ob = (sb_ref[...] + gb_ref[...]) * dinv
        o_ref[...] = jnp.concatenate([oa, ob], axis=1) + b1_ref[...]

    return pl.pallas_call(
        body,
        grid=(10,),
        in_specs=[
            pl.BlockSpec((1000, H), lambda i: (i, 0)),
            pl.BlockSpec((1000, H), lambda i: (i, 0)),
            pl.BlockSpec((1000, H), lambda i: (i, 0)),
            pl.BlockSpec((1000, H), lambda i: (i, 0)),
            pl.BlockSpec((1000, H), lambda i: (i, 0)),
            pl.BlockSpec((1000, H), lambda i: (i, 0)),
            pl.BlockSpec((1, D), lambda i: (0, 0)),
        ],
        out_specs=pl.BlockSpec((1000, D), lambda i: (i, 0)),
        out_shape=jax.ShapeDtypeStruct((N, D), jnp.float32),
    )(s1a, s1b, g1a, g1b, dega, degb, b1r)


def kernel(x, edge_index, x_batch, W0, b0, W1, b1):
    del x_batch
    src = edge_index[0]
    dst = edge_index[1]

    dega, degb = _sc_degree(dst)
    h0 = _tc_matmul(x, W0)
    ga, gb = _tc_scale(h0, dega, degb)
    sa, sb = _sc_scatter(ga, gb, src, dst)
    g1a, g1b = _tc_mid(sa, sb, ga, gb, dega, degb, b0, W1)
    s1a, s1b = _sc_scatter(g1a, g1b, src, dst)
    return _tc_out(s1a, s1b, g1a, g1b, dega, degb, b1)
